# baseline (speedup 1.0000x reference)
import jax
import jax.numpy as jnp
from jax import lax
from jax.experimental import pallas as pl
from jax.experimental.pallas import tpu as pltpu
from jax.experimental.pallas import tpu_sc as plsc

N = 10000
D = 128
E = 320000

NC = 2
NS = 16
NW = NC * NS
LANES = 16

C2 = 64
CPW2 = 160
CPT = CPW2 * NC
BLK2 = 32
NBUF = 2
E_PAD = NW * CPW2 * C2
NCHUNKS = E_PAD // C2

N_ACC = 10240
RPT = N_ACC // NS


def _newton_rsqrt(d):
    dsafe = jnp.maximum(d, 1.0)
    y = jnp.full_like(dsafe, 0.70710678)
    for k in range(1, 10):
        y = jnp.where(dsafe >= float(4.0 ** k), 0.70710678 * 2.0 ** (-k), y)
    for _ in range(6):
        y = y * (1.5 - 0.5 * dsafe * y * y)
    return jnp.where(d > 0.5, y, 0.0)


def _scale_rows(buf, dv, nrows, off):
    @pl.loop(0, nrows)
    def _(j):
        idx = jnp.full((LANES,), off + j, jnp.int32)
        dsp = plsc.load_gather(dv, [idx])
        for k in range(D // LANES):
            sl = pl.ds(k * LANES, LANES)
            buf[j, sl] = buf[j, sl] * dsp


def _sc_body(rows_hbm, cols_hbm, x_hbm, s1p0_hbm, s1p1_hbm, y_hbm,
             ridx, cidx, vals, rowbuf0, rowbuf1, sbuf, zb, dv, xv,
             deg_sh, acc_sh, dsem, gsem):
    c = lax.axis_index("c")
    s = lax.axis_index("s")
    w = s * NC + c
    lane = lax.iota(jnp.int32, LANES)
    rbufs = (rowbuf0, rowbuf1)
    outs = (s1p0_hbm, s1p1_hbm)

    @pl.loop(0, RPT // LANES)
    def _(i):
        zb[pl.ds(i * LANES, LANES)] = jnp.zeros((LANES,), jnp.float32)

    @pl.loop(0, 64)
    def _(i):
        for k in range(D // LANES):
            sbuf[i, pl.ds(k * LANES, LANES)] = jnp.zeros((LANES,), jnp.float32)

    pltpu.sync_copy(zb, deg_sh.at[pl.ds(s * RPT, RPT)])

    @pl.loop(0, RPT // 64)
    def _(b):
        pltpu.sync_copy(sbuf, acc_sh.at[pl.ds(s * RPT + b * 64, 64)])

    plsc.subcore_barrier()

    @pl.loop(0, CPT // BLK2)
    def _(g):
        base = s * CPT + g * BLK2
        pltpu.sync_copy(rows_hbm.at[pl.ds(base, BLK2)], ridx)
        pltpu.sync_copy(cols_hbm.at[pl.ds(base, BLK2)], cidx)

        for ch in range(BLK2):
            for k in range(C2 // LANES):
                sl = pl.ds(k * LANES, LANES)
                r = ridx[ch, sl]
                cc = cidx[ch, sl]
                vals[ch, sl] = jnp.where(r != cc, 1.0, 0.0).astype(jnp.float32)

        for ch in range(BLK2):
            pltpu.async_copy(vals.at[ch], deg_sh.at[ridx.at[ch]], dsem,
                             add=True)
        for ch in range(BLK2):
            pltpu.make_async_copy(vals.at[ch], deg_sh.at[ridx.at[ch]],
                                  dsem).wait()

    plsc.subcore_barrier()

    pltpu.sync_copy(deg_sh.at[pl.ds(s * RPT, RPT)], dv)

    @pl.loop(0, RPT // LANES)
    def _(i):
        sl = pl.ds(i * LANES, LANES)
        dv[sl] = _newton_rsqrt(dv[sl])

    @pl.loop(0, RPT // 64)
    def _(b):
        start = jnp.minimum(s * RPT + b * 64, N - 64)
        pltpu.sync_copy(x_hbm.at[pl.ds(start, 64)], xv)
        _scale_rows(xv, dv, 64, start - s * RPT)
        pltpu.sync_copy(xv, y_hbm.at[pl.ds(start, 64)])

    plsc.subcore_barrier()

    @pl.loop(0, CPW2 // BLK2)
    def _(g):
        base = w * CPW2 + g * BLK2
        pltpu.sync_copy(rows_hbm.at[pl.ds(base, BLK2)], ridx)
        pltpu.sync_copy(cols_hbm.at[pl.ds(base, BLK2)], cidx)

        for ch in range(BLK2):
            for k in range(C2 // LANES):
                sl = pl.ds(k * LANES, LANES)
                r = ridx[ch, sl]
                cc = cidx[ch, sl]
                cidx[ch, sl] = jnp.where(r == cc, N + lane, cc)

        for p in range(NBUF - 1):
            pltpu.async_copy(y_hbm.at[ridx.at[p]], rbufs[p], gsem)
        for ch in range(BLK2):
            cur = rbufs[ch % NBUF]
            pltpu.make_async_copy(y_hbm.at[ridx.at[ch]], cur, gsem).wait()
            pltpu.sync_copy(cur, acc_sh.at[cidx.at[ch]], add=True)
            if ch + NBUF - 1 < BLK2:
                pltpu.async_copy(
                    y_hbm.at[ridx.at[ch + NBUF - 1]],
                    rbufs[(ch + NBUF - 1) % NBUF], gsem)

    plsc.subcore_barrier()

    @pl.loop(0, RPT // 64)
    def _(b):
        pltpu.sync_copy(acc_sh.at[pl.ds(s * RPT + b * 64, 64)], sbuf)
        _scale_rows(sbuf, dv, 64, b * 64)
        for cc in range(NC):
            @pl.when(c == cc)
            def _():
                pltpu.sync_copy(
                    sbuf, outs[cc].at[pl.ds(s * RPT + b * 64, 64)])


_SC_MESH = plsc.VectorSubcoreMesh(core_axis_name="c", subcore_axis_name="s")
_SC_PARAMS = pltpu.CompilerParams(needs_layout_passes=False)

_sc_aggregate = pl.kernel(
    _sc_body,
    compiler_params=_SC_PARAMS,
    out_type=(
        jax.ShapeDtypeStruct((N_ACC, D), jnp.float32),
        jax.ShapeDtypeStruct((N_ACC, D), jnp.float32),
        jax.ShapeDtypeStruct((N, D), jnp.float32),
    ),
    mesh=_SC_MESH,
    scratch_types=[
        pltpu.VMEM((BLK2, C2), jnp.int32),
        pltpu.VMEM((BLK2, C2), jnp.int32),
        pltpu.VMEM((BLK2, C2), jnp.float32),
        pltpu.VMEM((C2, D), jnp.float32),
        pltpu.VMEM((C2, D), jnp.float32),
        pltpu.VMEM((64, D), jnp.float32),
        pltpu.VMEM((RPT,), jnp.float32),
        pltpu.VMEM((RPT,), jnp.float32),
        pltpu.VMEM((64, D), jnp.float32),
        pltpu.VMEM_SHARED((N_ACC,), jnp.float32),
        pltpu.VMEM_SHARED((N_ACC, D), jnp.float32),
        pltpu.SemaphoreType.DMA,
        pltpu.SemaphoreType.DMA,
    ],
)


def _final_body(s0_ref, s1_ref, x_ref, w1_ref, w2_ref, b_ref, o_ref):
    s1 = s0_ref[...] + s1_ref[...]
    x = x_ref[...]
    m = (jnp.dot(x + s1, w1_ref[...], preferred_element_type=jnp.float32)
         + jnp.dot(x * s1, w2_ref[...], preferred_element_type=jnp.float32)
         + b_ref[...])
    ss = jnp.sum(m * m, axis=1, keepdims=True)
    o_ref[...] = m / jnp.maximum(jnp.sqrt(ss), 1e-12)


_BN = 1000

_final = pl.pallas_call(
    _final_body,
    out_shape=jax.ShapeDtypeStruct((N, D), jnp.float32),
    grid=(N // _BN,),
    in_specs=[
        pl.BlockSpec((_BN, D), lambda i: (i, 0)),
        pl.BlockSpec((_BN, D), lambda i: (i, 0)),
        pl.BlockSpec((_BN, D), lambda i: (i, 0)),
        pl.BlockSpec((D, D), lambda i: (0, 0)),
        pl.BlockSpec((D, D), lambda i: (0, 0)),
        pl.BlockSpec((1, D), lambda i: (0, 0)),
    ],
    out_specs=pl.BlockSpec((_BN, D), lambda i: (i, 0)),
)


@jax.jit
def kernel(x, edge_index, W1, W2, b):
    row = edge_index[0].astype(jnp.int32)
    col = edge_index[1].astype(jnp.int32)
    pad = E_PAD - E
    pidx = (jnp.arange(pad, dtype=jnp.int32) % N)
    rows2c = jnp.concatenate([row, pidx]).reshape(NCHUNKS, C2)
    cols2c = jnp.concatenate([col, pidx]).reshape(NCHUNKS, C2)

    s0, s1, _y = _sc_aggregate(rows2c, cols2c, x)
    return _final(s0, s1, x, W1, W2, b.reshape(1, D))


# speedup vs baseline: 1.7515x; 1.7515x over previous
import functools

import jax
import jax.numpy as jnp
from jax import lax
from jax.experimental import pallas as pl
from jax.experimental.pallas import tpu as pltpu
from jax.experimental.pallas import tpu_sc as plsc

N = 10000
D = 128
E = 320000

NC = 2
NS = 16
NW = NC * NS
LANES = 16

CHUNK = 128
CPW = 80
C2 = 64
CPW2 = 160
BLK2 = 32
NBUF = 3
E_PAD = NW * CPW * CHUNK
NCHUNKS = E_PAD // CHUNK

N_ACC = 10240
RPT = N_ACC // NS
YPW = 320

def _newton_rsqrt(d):
    dsafe = jnp.maximum(d, 1.0)
    y = jnp.full_like(dsafe, 0.70710678)
    for k in range(1, 10):
        y = jnp.where(dsafe >= float(4.0 ** k), 0.70710678 * 2.0 ** (-k), y)
    for _ in range(6):
        y = y * (1.5 - 0.5 * dsafe * y * y)
    return jnp.where(d > 0.5, y, 0.0)


def _scale_rows(buf, dv, nrows, off):
    @pl.loop(0, nrows)
    def _(j):
        idx = jnp.full((LANES,), off + j, jnp.int32)
        dsp = plsc.load_gather(dv, [idx])
        for k in range(D // LANES):
            sl = pl.ds(k * LANES, LANES)
            buf[j, sl] = buf[j, sl] * dsp


def _deg_dis_y_body(rows_hbm, cols_hbm, x_hbm, y_hbm, dis_hbm,
                    ridx, cidx, vals, zb, dv, xv, deg_sh, dsem):
    c = lax.axis_index("c")
    s = lax.axis_index("s")
    w = s * NC + c

    @pl.loop(0, RPT // LANES)
    def _(i):
        zb[pl.ds(i * LANES, LANES)] = jnp.zeros((LANES,), jnp.float32)

    pltpu.sync_copy(zb, deg_sh.at[pl.ds(s * RPT, RPT)])
    plsc.subcore_barrier()

    for half in range(2):
        base = s * 2 * CPW + half * CPW

        pltpu.sync_copy(rows_hbm.at[pl.ds(base, CPW)], ridx)
        pltpu.sync_copy(cols_hbm.at[pl.ds(base, CPW)], cidx)

        @pl.loop(0, CPW)
        def _(ch):
            for k in range(CHUNK // LANES):
                sl = pl.ds(k * LANES, LANES)
                r = ridx[ch, sl]
                cc = cidx[ch, sl]
                vals[ch, sl] = jnp.where(r != cc, 1.0, 0.0).astype(jnp.float32)

        @pl.loop(0, CPW // 16)
        def _(g):
            for j in range(16):
                pltpu.async_copy(
                    vals.at[g * 16 + j], deg_sh.at[ridx.at[g * 16 + j]],
                    dsem, add=True)
            for j in range(16):
                pltpu.make_async_copy(
                    vals.at[g * 16 + j], deg_sh.at[ridx.at[g * 16 + j]],
                    dsem).wait()

    plsc.subcore_barrier()

    pltpu.sync_copy(deg_sh.at[pl.ds(s * RPT, RPT)], dv)

    @pl.loop(0, RPT // LANES)
    def _(i):
        sl = pl.ds(i * LANES, LANES)
        dv[sl] = _newton_rsqrt(dv[sl])

    @pl.when(c == 0)
    def _():
        pltpu.sync_copy(dv, dis_hbm.at[pl.ds(s * RPT, RPT)])

    start = jnp.minimum(w * YPW, N - YPW)
    off = start - s * RPT
    pltpu.sync_copy(x_hbm.at[pl.ds(start, YPW)], xv)
    _scale_rows(xv, dv, YPW, off)
    pltpu.sync_copy(xv, y_hbm.at[pl.ds(start, YPW)])


def _scatter_body(rows_hbm, cols_hbm, y_hbm, dis_hbm, s1p0_hbm, s1p1_hbm,
                  ridx, cidx2, rowbuf0, rowbuf1, rowbuf2,
                  sbuf, dv, acc_sh, gsem, ssem0, ssem1, ssem2):
    rbufs = (rowbuf0, rowbuf1, rowbuf2)
    ssems = (ssem0, ssem1, ssem2)
    outs = (s1p0_hbm, s1p1_hbm)
    c = lax.axis_index("c")
    s = lax.axis_index("s")
    w = s * NC + c
    lane = lax.iota(jnp.int32, LANES)

    @pl.loop(0, 64)
    def _(i):
        for k in range(D // LANES):
            sbuf[i, pl.ds(k * LANES, LANES)] = jnp.zeros((LANES,), jnp.float32)

    @pl.loop(0, RPT // 64)
    def _(b):
        pltpu.sync_copy(sbuf, acc_sh.at[pl.ds(s * RPT + b * 64, 64)])

    plsc.subcore_barrier()

    base = w * CPW2

    @pl.loop(0, CPW2 // BLK2)
    def _(g):
        pltpu.sync_copy(rows_hbm.at[pl.ds(base + g * BLK2, BLK2)], ridx)
        pltpu.sync_copy(cols_hbm.at[pl.ds(base + g * BLK2, BLK2)], cidx2)

        for ch in range(BLK2):
            for k in range(C2 // LANES):
                sl = pl.ds(k * LANES, LANES)
                r = ridx[ch, sl]
                cc = cidx2[ch, sl]
                cidx2[ch, sl] = jnp.where(r == cc, N + lane, cc)

        for p in range(NBUF - 1):
            pltpu.async_copy(y_hbm.at[ridx.at[p]], rbufs[p], gsem)
        for ch in range(BLK2):
            b = ch % NBUF
            cur = rbufs[b]
            pltpu.make_async_copy(y_hbm.at[ridx.at[ch]], cur, gsem).wait()
            pltpu.async_copy(cur, acc_sh.at[cidx2.at[ch]], ssems[b],
                             add=True)
            nb = (ch + NBUF - 1) % NBUF
            if ch + NBUF - 1 < BLK2:
                if ch >= 1:
                    pltpu.make_async_copy(
                        rbufs[nb], acc_sh.at[cidx2.at[ch - 1]],
                        ssems[nb]).wait()
                pltpu.async_copy(
                    y_hbm.at[ridx.at[ch + NBUF - 1]], rbufs[nb], gsem)
        for ch in range(BLK2 - NBUF, BLK2):
            b = ch % NBUF
            pltpu.make_async_copy(
                rbufs[b], acc_sh.at[cidx2.at[ch]], ssems[b]).wait()

    plsc.subcore_barrier()

    pltpu.sync_copy(dis_hbm.at[pl.ds(s * RPT, RPT)], dv)

    @pl.loop(0, RPT // 64)
    def _(b):
        pltpu.sync_copy(acc_sh.at[pl.ds(s * RPT + b * 64, 64)], sbuf)
        _scale_rows(sbuf, dv, 64, b * 64)
        for cc in range(NC):
            @pl.when(c == cc)
            def _():
                pltpu.sync_copy(
                    sbuf, outs[cc].at[pl.ds(s * RPT + b * 64, 64)])


_SC_MESH = plsc.VectorSubcoreMesh(core_axis_name="c", subcore_axis_name="s")
_SC_PARAMS = pltpu.CompilerParams(needs_layout_passes=False)

_deg_dis_y = pl.kernel(
    _deg_dis_y_body,
    compiler_params=_SC_PARAMS,
    out_type=(
        jax.ShapeDtypeStruct((N, D), jnp.float32),
        jax.ShapeDtypeStruct((N_ACC,), jnp.float32),
    ),
    mesh=_SC_MESH,
    scratch_types=[
        pltpu.VMEM((CPW, CHUNK), jnp.int32),
        pltpu.VMEM((CPW, CHUNK), jnp.int32),
        pltpu.VMEM((CPW, CHUNK), jnp.float32),
        pltpu.VMEM((RPT,), jnp.float32),
        pltpu.VMEM((RPT,), jnp.float32),
        pltpu.VMEM((YPW, D), jnp.float32),
        pltpu.VMEM_SHARED((N_ACC,), jnp.float32),
        pltpu.SemaphoreType.DMA,
    ],
)

_scatter = pl.kernel(
    _scatter_body,
    compiler_params=_SC_PARAMS,
    out_type=(
        jax.ShapeDtypeStruct((N_ACC, D), jnp.float32),
        jax.ShapeDtypeStruct((N_ACC, D), jnp.float32),
    ),
    mesh=_SC_MESH,
    scratch_types=[
        pltpu.VMEM((BLK2, C2), jnp.int32),
        pltpu.VMEM((BLK2, C2), jnp.int32),
        pltpu.VMEM((C2, D), jnp.float32),
        pltpu.VMEM((C2, D), jnp.float32),
        pltpu.VMEM((C2, D), jnp.float32),
        pltpu.VMEM((64, D), jnp.float32),
        pltpu.VMEM((RPT,), jnp.float32),
        pltpu.VMEM_SHARED((N_ACC, D), jnp.float32),
        pltpu.SemaphoreType.DMA,
        pltpu.SemaphoreType.DMA,
        pltpu.SemaphoreType.DMA,
        pltpu.SemaphoreType.DMA,
    ],
)


def _final_body(s0_ref, s1_ref, x_ref, w1_ref, w2_ref, b_ref, o_ref):
    s1 = s0_ref[...] + s1_ref[...]
    x = x_ref[...]
    m = (jnp.dot(x + s1, w1_ref[...], preferred_element_type=jnp.float32)
         + jnp.dot(x * s1, w2_ref[...], preferred_element_type=jnp.float32)
         + b_ref[...])
    ss = jnp.sum(m * m, axis=1, keepdims=True)
    o_ref[...] = m / jnp.maximum(jnp.sqrt(ss), 1e-12)


_BN = 1000

_final = pl.pallas_call(
    _final_body,
    out_shape=jax.ShapeDtypeStruct((N, D), jnp.float32),
    grid=(N // _BN,),
    in_specs=[
        pl.BlockSpec((_BN, D), lambda i: (i, 0)),
        pl.BlockSpec((_BN, D), lambda i: (i, 0)),
        pl.BlockSpec((_BN, D), lambda i: (i, 0)),
        pl.BlockSpec((D, D), lambda i: (0, 0)),
        pl.BlockSpec((D, D), lambda i: (0, 0)),
        pl.BlockSpec((1, D), lambda i: (0, 0)),
    ],
    out_specs=pl.BlockSpec((_BN, D), lambda i: (i, 0)),
)


@jax.jit
def kernel(x, edge_index, W1, W2, b):
    row = edge_index[0].astype(jnp.int32)
    col = edge_index[1].astype(jnp.int32)
    pad = E_PAD - E
    pidx = (jnp.arange(pad, dtype=jnp.int32) % N)
    rows2d = jnp.concatenate([row, pidx]).reshape(NCHUNKS, CHUNK)
    cols2d = jnp.concatenate([col, pidx]).reshape(NCHUNKS, CHUNK)

    y, dis = _deg_dis_y(rows2d, cols2d, x)
    rows2c = rows2d.reshape(E_PAD // C2, C2)
    cols2c = cols2d.reshape(E_PAD // C2, C2)
    s0, s1 = _scatter(rows2c, cols2c, y, dis)
    return _final(s0, s1, x, W1, W2, b.reshape(1, D))
